# baseline (speedup 1.0000x reference)
import dataclasses
import functools

import jax
import jax.numpy as jnp
from jax import lax
from jax.experimental import pallas as pl
from jax.experimental.pallas import tpu as pltpu
from jax.experimental.pallas import tpu_sc as plsc

NUM_WORKERS = 32
LANES = 16
GROUP_ROWS = 16
NCHUNKS = 4
MM_BM = 512


def _densify_body(nnz_p, k, chunk_base_gid, groups,
                  vals_hbm, cols_hbm, w_hbm,
                  vals_v0, cols_v0, vals_v1, cols_v1, buf, bbuf_a, bbuf_b,
                  sem_a, sem_b, sem_in):
    wid = lax.axis_index("s") * 2 + lax.axis_index("c")
    group_nnz = nnz_p * GROUP_ROWS
    half = GROUP_ROWS * k // 2
    lane_base = lax.iota(jnp.int32, LANES) * k
    strip_base = lax.iota(jnp.int32, LANES) * nnz_p
    zeros16 = jnp.zeros((LANES,), jnp.float32)

    @pl.loop(0, GROUP_ROWS * k, step=LANES, unroll=4)
    def _(j):
        buf[pl.ds(j, LANES)] = zeros16

    def fetch(gid, vals_v, cols_v):
        base = gid * group_nnz
        pltpu.async_copy(cols_hbm.at[pl.ds(base, group_nnz)], cols_v, sem_in)
        pltpu.async_copy(vals_hbm.at[pl.ds(base, group_nnz)], vals_v, sem_in)

    def convert_half(bbuf, offs):
        @pl.loop(0, half, step=2 * LANES, unroll=4)
        def _(j):
            a = buf[pl.ds(offs + j, LANES)]
            b = buf[pl.ds(offs + j + LANES, LANES)]
            bbuf[pl.ds(j, 2 * LANES)] = plsc.pack(
                a, b, format=plsc.PackFormat.INTERLEAVED)

    def handle(g, cur, nxt):
        vals_v, cols_v = cur
        local_gid = wid * groups + g
        gid = chunk_base_gid + local_gid
        pltpu.make_async_copy(
            cols_hbm.at[pl.ds(0, group_nnz)], cols_v, sem_in).wait()
        pltpu.make_async_copy(
            vals_hbm.at[pl.ds(0, group_nnz)], vals_v, sem_in).wait()

        @pl.when(g + 1 < groups)
        def _():
            fetch(gid + 1, *nxt)

        @pl.loop(0, nnz_p, unroll=4)
        def _(j):
            strip = strip_base + j
            cv = plsc.load_gather(cols_v, [strip])
            vv = plsc.load_gather(vals_v, [strip])
            plsc.addupdate_scatter(buf, [lane_base + cv], vv)

        out_base = local_gid * GROUP_ROWS * k

        @pl.when(g > 0)
        def _():
            pltpu.make_async_copy(
                bbuf_a, w_hbm.at[pl.ds(out_base, half)], sem_a).wait()
        convert_half(bbuf_a, 0)
        pltpu.async_copy(bbuf_a, w_hbm.at[pl.ds(out_base, half)], sem_a)

        @pl.when(g > 0)
        def _():
            pltpu.make_async_copy(
                bbuf_b, w_hbm.at[pl.ds(out_base + half, half)], sem_b).wait()
        convert_half(bbuf_b, half)
        pltpu.async_copy(bbuf_b, w_hbm.at[pl.ds(out_base + half, half)], sem_b)

        @pl.loop(0, nnz_p, unroll=4)
        def _(j):
            cv = plsc.load_gather(cols_v, [strip_base + j])
            plsc.store_scatter(buf, [lane_base + cv], zeros16)

    set0 = (vals_v0, cols_v0)
    set1 = (vals_v1, cols_v1)
    fetch(chunk_base_gid + wid * groups, *set0)

    @pl.loop(0, groups // 2)
    def _(p):
        handle(2 * p, set0, set1)
        handle(2 * p + 1, set1, set0)

    last = (wid * groups + groups - 1) * GROUP_ROWS * k
    pltpu.make_async_copy(bbuf_a, w_hbm.at[pl.ds(last, half)], sem_a).wait()
    pltpu.make_async_copy(
        bbuf_b, w_hbm.at[pl.ds(last + half, half)], sem_b).wait()


def _densify_chunk(values_g, cols_g, nnz_p, chunk_m, k, chunk_base_gid):
    groups = chunk_m // NUM_WORKERS // GROUP_ROWS
    mesh = plsc.VectorSubcoreMesh(core_axis_name="c", subcore_axis_name="s")
    cp = pltpu.CompilerParams()
    if "needs_layout_passes" in pltpu.CompilerParams.__dataclass_fields__:
        cp = dataclasses.replace(cp, needs_layout_passes=False)
    half = GROUP_ROWS * k // 2
    group_nnz = nnz_p * GROUP_ROWS
    kern = pl.kernel(
        functools.partial(_densify_body, nnz_p, k, chunk_base_gid, groups),
        out_type=jax.ShapeDtypeStruct((chunk_m * k,), jnp.bfloat16),
        mesh=mesh,
        scratch_types=[
            pltpu.VMEM((group_nnz,), jnp.float32),
            pltpu.VMEM((group_nnz,), jnp.int32),
            pltpu.VMEM((group_nnz,), jnp.float32),
            pltpu.VMEM((group_nnz,), jnp.int32),
            pltpu.VMEM((GROUP_ROWS * k,), jnp.float32),
            pltpu.VMEM((half,), jnp.bfloat16),
            pltpu.VMEM((half,), jnp.bfloat16),
            pltpu.SemaphoreType.DMA,
            pltpu.SemaphoreType.DMA,
            pltpu.SemaphoreType.DMA,
        ],
        compiler_params=cp,
    )
    return kern(values_g, cols_g)


def _mm_first_body(w_ref, xb_ref, o_ref):
    o_ref[...] = lax.dot_general(
        w_ref[...], xb_ref[...], (((1,), (1,)), ((), ())),
        preferred_element_type=jnp.float32,
    )


def _mm_chain_body(w_ref, xb_ref, prev_ref, o_ref):
    del prev_ref
    o_ref[...] = lax.dot_general(
        w_ref[...], xb_ref[...], (((1,), (1,)), ((), ())),
        preferred_element_type=jnp.float32,
    )


def _matmul_chunk(w, xb, c, m_total, out_prev):
    chunk_m, k = w.shape
    seq = xb.shape[0]
    grid = (chunk_m // MM_BM,)
    blocks_before = c * (chunk_m // MM_BM)
    out_spec = pl.BlockSpec((MM_BM, seq), lambda i: (blocks_before + i, 0))
    in_specs = [
        pl.BlockSpec((MM_BM, k), lambda i: (i, 0)),
        pl.BlockSpec((seq, k), lambda i: (0, 0)),
    ]
    out_shape = jax.ShapeDtypeStruct((m_total, seq), jnp.float32)
    if out_prev is None:
        return pl.pallas_call(
            _mm_first_body, grid=grid, in_specs=in_specs,
            out_specs=out_spec, out_shape=out_shape,
        )(w, xb)
    return pl.pallas_call(
        _mm_chain_body, grid=grid,
        in_specs=in_specs + [
            pl.BlockSpec(memory_space=pltpu.MemorySpace.HBM)],
        out_specs=out_spec, out_shape=out_shape,
        input_output_aliases={2: 0},
    )(w, xb, out_prev)


def kernel(x, values, row_indices, row_offsets, column_indices):
    b, seq, k = x.shape
    m = row_offsets.shape[0] - 1
    nnz_p = values.shape[0] // m

    r = column_indices & 31
    cols_p = (column_indices & ~31) | ((r & 1) << 4) | (r >> 1)

    xb = x[0].astype(jnp.bfloat16)

    chunk_m = m // NCHUNKS
    out = None
    for c in range(NCHUNKS):
        wc = _densify_chunk(
            values, cols_p, nnz_p, chunk_m, k,
            c * chunk_m // GROUP_ROWS).reshape(chunk_m, k)
        out = _matmul_chunk(wc, xb, c, m, out)
    return out.reshape(b, m, seq)


# speedup vs baseline: 1.0232x; 1.0232x over previous
import dataclasses
import functools

import jax
import jax.numpy as jnp
from jax import lax
from jax.experimental import pallas as pl
from jax.experimental.pallas import tpu as pltpu
from jax.experimental.pallas import tpu_sc as plsc

NUM_WORKERS = 32
LANES = 16
GROUP_ROWS = 16
NCHUNKS = 4
MM_BM = 512


def _densify_body(nnz_p, k, chunk_base_gid, groups,
                  vals_hbm, cols_hbm, w_hbm,
                  vals_v0, cols_v0, vals_v1, cols_v1, buf, bbuf_a, bbuf_b,
                  sem_a, sem_b, sem_in):
    wid = lax.axis_index("s") * 2 + lax.axis_index("c")
    group_nnz = nnz_p * GROUP_ROWS
    half = GROUP_ROWS * k // 2
    lane_base = lax.iota(jnp.int32, LANES) * k
    strip_base = lax.iota(jnp.int32, LANES) * nnz_p
    zeros16 = jnp.zeros((LANES,), jnp.float32)

    @pl.loop(0, GROUP_ROWS * k, step=LANES, unroll=4)
    def _(j):
        buf[pl.ds(j, LANES)] = zeros16

    def fetch(gid, vals_v, cols_v):
        base = gid * group_nnz
        pltpu.async_copy(cols_hbm.at[pl.ds(base, group_nnz)], cols_v, sem_in)
        pltpu.async_copy(vals_hbm.at[pl.ds(base, group_nnz)], vals_v, sem_in)

    def convert_half(bbuf, offs):
        @pl.loop(0, half, step=2 * LANES, unroll=4)
        def _(j):
            a = buf[pl.ds(offs + j, LANES)]
            b = buf[pl.ds(offs + j + LANES, LANES)]
            bbuf[pl.ds(j, 2 * LANES)] = plsc.pack(
                a, b, format=plsc.PackFormat.INTERLEAVED)
            buf[pl.ds(offs + j, LANES)] = zeros16
            buf[pl.ds(offs + j + LANES, LANES)] = zeros16

    def handle(g, cur, nxt):
        vals_v, cols_v = cur
        local_gid = wid * groups + g
        gid = chunk_base_gid + local_gid
        pltpu.make_async_copy(
            cols_hbm.at[pl.ds(0, group_nnz)], cols_v, sem_in).wait()
        pltpu.make_async_copy(
            vals_hbm.at[pl.ds(0, group_nnz)], vals_v, sem_in).wait()

        @pl.when(g + 1 < groups)
        def _():
            fetch(gid + 1, *nxt)

        @pl.loop(0, nnz_p, unroll=4)
        def _(j):
            strip = strip_base + j
            cv = plsc.load_gather(cols_v, [strip])
            vv = plsc.load_gather(vals_v, [strip])
            plsc.addupdate_scatter(buf, [lane_base + cv], vv)

        out_base = local_gid * GROUP_ROWS * k

        @pl.when(g > 0)
        def _():
            pltpu.make_async_copy(
                bbuf_a, w_hbm.at[pl.ds(out_base, half)], sem_a).wait()
        convert_half(bbuf_a, 0)
        pltpu.async_copy(bbuf_a, w_hbm.at[pl.ds(out_base, half)], sem_a)

        @pl.when(g > 0)
        def _():
            pltpu.make_async_copy(
                bbuf_b, w_hbm.at[pl.ds(out_base + half, half)], sem_b).wait()
        convert_half(bbuf_b, half)
        pltpu.async_copy(bbuf_b, w_hbm.at[pl.ds(out_base + half, half)], sem_b)

    set0 = (vals_v0, cols_v0)
    set1 = (vals_v1, cols_v1)
    fetch(chunk_base_gid + wid * groups, *set0)

    @pl.loop(0, groups // 2)
    def _(p):
        handle(2 * p, set0, set1)
        handle(2 * p + 1, set1, set0)

    last = (wid * groups + groups - 1) * GROUP_ROWS * k
    pltpu.make_async_copy(bbuf_a, w_hbm.at[pl.ds(last, half)], sem_a).wait()
    pltpu.make_async_copy(
        bbuf_b, w_hbm.at[pl.ds(last + half, half)], sem_b).wait()


def _densify_chunk(values_g, cols_g, nnz_p, chunk_m, k, chunk_base_gid):
    groups = chunk_m // NUM_WORKERS // GROUP_ROWS
    mesh = plsc.VectorSubcoreMesh(core_axis_name="c", subcore_axis_name="s")
    cp = pltpu.CompilerParams()
    if "needs_layout_passes" in pltpu.CompilerParams.__dataclass_fields__:
        cp = dataclasses.replace(cp, needs_layout_passes=False)
    half = GROUP_ROWS * k // 2
    group_nnz = nnz_p * GROUP_ROWS
    kern = pl.kernel(
        functools.partial(_densify_body, nnz_p, k, chunk_base_gid, groups),
        out_type=jax.ShapeDtypeStruct((chunk_m * k,), jnp.bfloat16),
        mesh=mesh,
        scratch_types=[
            pltpu.VMEM((group_nnz,), jnp.float32),
            pltpu.VMEM((group_nnz,), jnp.int32),
            pltpu.VMEM((group_nnz,), jnp.float32),
            pltpu.VMEM((group_nnz,), jnp.int32),
            pltpu.VMEM((GROUP_ROWS * k,), jnp.float32),
            pltpu.VMEM((half,), jnp.bfloat16),
            pltpu.VMEM((half,), jnp.bfloat16),
            pltpu.SemaphoreType.DMA,
            pltpu.SemaphoreType.DMA,
            pltpu.SemaphoreType.DMA,
        ],
        compiler_params=cp,
    )
    return kern(values_g, cols_g)


def _mm_first_body(w_ref, xb_ref, o_ref):
    o_ref[...] = lax.dot_general(
        w_ref[...], xb_ref[...], (((1,), (1,)), ((), ())),
        preferred_element_type=jnp.float32,
    )


def _mm_chain_body(w_ref, xb_ref, prev_ref, o_ref):
    del prev_ref
    o_ref[...] = lax.dot_general(
        w_ref[...], xb_ref[...], (((1,), (1,)), ((), ())),
        preferred_element_type=jnp.float32,
    )


def _matmul_chunk(w, xb, c, m_total, out_prev):
    chunk_m, k = w.shape
    seq = xb.shape[0]
    grid = (chunk_m // MM_BM,)
    blocks_before = c * (chunk_m // MM_BM)
    out_spec = pl.BlockSpec((MM_BM, seq), lambda i: (blocks_before + i, 0))
    in_specs = [
        pl.BlockSpec((MM_BM, k), lambda i: (i, 0)),
        pl.BlockSpec((seq, k), lambda i: (0, 0)),
    ]
    out_shape = jax.ShapeDtypeStruct((m_total, seq), jnp.float32)
    if out_prev is None:
        return pl.pallas_call(
            _mm_first_body, grid=grid, in_specs=in_specs,
            out_specs=out_spec, out_shape=out_shape,
        )(w, xb)
    return pl.pallas_call(
        _mm_chain_body, grid=grid,
        in_specs=in_specs + [
            pl.BlockSpec(memory_space=pltpu.MemorySpace.HBM)],
        out_specs=out_spec, out_shape=out_shape,
        input_output_aliases={2: 0},
    )(w, xb, out_prev)


def kernel(x, values, row_indices, row_offsets, column_indices):
    b, seq, k = x.shape
    m = row_offsets.shape[0] - 1
    nnz_p = values.shape[0] // m

    r = column_indices & 31
    cols_p = (column_indices & ~31) | ((r & 1) << 4) | (r >> 1)

    xb = x[0].astype(jnp.bfloat16)

    chunk_m = m // NCHUNKS
    ws = [
        _densify_chunk(
            values, cols_p, nnz_p, chunk_m, k,
            c * chunk_m // GROUP_ROWS).reshape(chunk_m, k)
        for c in range(NCHUNKS)
    ]
    out = None
    for c in range(NCHUNKS):
        out = _matmul_chunk(ws[c], xb, c, m, out)
    return out.reshape(b, m, seq)


# speedup vs baseline: 1.1491x; 1.1231x over previous
import dataclasses
import functools

import jax
import jax.numpy as jnp
from jax import lax
from jax.experimental import pallas as pl
from jax.experimental.pallas import tpu as pltpu
from jax.experimental.pallas import tpu_sc as plsc

NUM_WORKERS = 32
LANES = 16
GROUP_ROWS = 16
NCHUNKS = 4
MM_BM = 512


def _densify_body(nnz_p, k, chunk_base_gid, groups,
                  vals_hbm, cols_hbm, w_hbm,
                  vals_v0, cols_v0, vals_v1, cols_v1, buf, bbuf_a, bbuf_b,
                  sem_a, sem_b, sem_in):
    wid = lax.axis_index("s") * 2 + lax.axis_index("c")
    group_nnz = nnz_p * GROUP_ROWS
    half = GROUP_ROWS * k // 2
    lane_base = lax.iota(jnp.int32, LANES) * k
    strip_base = lax.iota(jnp.int32, LANES) * nnz_p
    zeros16 = jnp.zeros((LANES,), jnp.float32)

    @pl.loop(0, GROUP_ROWS * k, step=LANES, unroll=4)
    def _(j):
        buf[pl.ds(j, LANES)] = zeros16

    def fetch(gid, vals_v, cols_v):
        base = gid * group_nnz
        pltpu.async_copy(cols_hbm.at[pl.ds(base, group_nnz)], cols_v, sem_in)
        pltpu.async_copy(vals_hbm.at[pl.ds(base, group_nnz)], vals_v, sem_in)

    def convert_half(bbuf, offs):
        @pl.loop(0, half, step=2 * LANES, unroll=4)
        def _(j):
            a = buf[pl.ds(offs + j, LANES)]
            b = buf[pl.ds(offs + j + LANES, LANES)]
            bbuf[pl.ds(j, 2 * LANES)] = plsc.pack(
                a, b, format=plsc.PackFormat.INTERLEAVED)
            buf[pl.ds(offs + j, LANES)] = zeros16
            buf[pl.ds(offs + j + LANES, LANES)] = zeros16

    def handle(g, cur, nxt):
        vals_v, cols_v = cur
        local_gid = wid * groups + g
        gid = chunk_base_gid + local_gid
        pltpu.make_async_copy(
            cols_hbm.at[pl.ds(0, group_nnz)], cols_v, sem_in).wait()
        pltpu.make_async_copy(
            vals_hbm.at[pl.ds(0, group_nnz)], vals_v, sem_in).wait()

        @pl.when(g + 1 < groups)
        def _():
            fetch(gid + 1, *nxt)

        @pl.loop(0, nnz_p, unroll=4)
        def _(j):
            strip = strip_base + j
            cv = plsc.load_gather(cols_v, [strip])
            vv = plsc.load_gather(vals_v, [strip])
            plsc.addupdate_scatter(buf, [lane_base + cv], vv)

        out_base = local_gid * GROUP_ROWS * k

        @pl.when(g > 0)
        def _():
            pltpu.make_async_copy(
                bbuf_a, w_hbm.at[pl.ds(out_base, half)], sem_a).wait()
        convert_half(bbuf_a, 0)
        pltpu.async_copy(bbuf_a, w_hbm.at[pl.ds(out_base, half)], sem_a)

        @pl.when(g > 0)
        def _():
            pltpu.make_async_copy(
                bbuf_b, w_hbm.at[pl.ds(out_base + half, half)], sem_b).wait()
        convert_half(bbuf_b, half)
        pltpu.async_copy(bbuf_b, w_hbm.at[pl.ds(out_base + half, half)], sem_b)

    set0 = (vals_v0, cols_v0)
    set1 = (vals_v1, cols_v1)
    fetch(chunk_base_gid + wid * groups, *set0)

    @pl.loop(0, groups // 2)
    def _(p):
        handle(2 * p, set0, set1)
        handle(2 * p + 1, set1, set0)

    last = (wid * groups + groups - 1) * GROUP_ROWS * k
    pltpu.make_async_copy(bbuf_a, w_hbm.at[pl.ds(last, half)], sem_a).wait()
    pltpu.make_async_copy(
        bbuf_b, w_hbm.at[pl.ds(last + half, half)], sem_b).wait()


def _densify_chunk(values_g, cols_g, nnz_p, chunk_m, k, chunk_base_gid):
    groups = chunk_m // NUM_WORKERS // GROUP_ROWS
    mesh = plsc.VectorSubcoreMesh(core_axis_name="c", subcore_axis_name="s")
    cp = pltpu.CompilerParams()
    if "needs_layout_passes" in pltpu.CompilerParams.__dataclass_fields__:
        cp = dataclasses.replace(cp, needs_layout_passes=False)
    half = GROUP_ROWS * k // 2
    group_nnz = nnz_p * GROUP_ROWS
    kern = pl.kernel(
        functools.partial(_densify_body, nnz_p, k, chunk_base_gid, groups),
        out_type=jax.ShapeDtypeStruct((chunk_m * k,), jnp.bfloat16),
        mesh=mesh,
        scratch_types=[
            pltpu.VMEM((group_nnz,), jnp.float32),
            pltpu.VMEM((group_nnz,), jnp.int32),
            pltpu.VMEM((group_nnz,), jnp.float32),
            pltpu.VMEM((group_nnz,), jnp.int32),
            pltpu.VMEM((GROUP_ROWS * k,), jnp.float32),
            pltpu.VMEM((half,), jnp.bfloat16),
            pltpu.VMEM((half,), jnp.bfloat16),
            pltpu.SemaphoreType.DMA,
            pltpu.SemaphoreType.DMA,
            pltpu.SemaphoreType.DMA,
        ],
        compiler_params=cp,
    )
    return kern(values_g, cols_g)


def _mm_first_body(k, w_ref, xb_ref, o_ref):
    w = w_ref[...].reshape(MM_BM, k)
    o_ref[...] = lax.dot_general(
        w, xb_ref[...], (((1,), (1,)), ((), ())),
        preferred_element_type=jnp.float32,
    )


def _mm_chain_body(k, w_ref, xb_ref, prev_ref, o_ref):
    del prev_ref
    w = w_ref[...].reshape(MM_BM, k)
    o_ref[...] = lax.dot_general(
        w, xb_ref[...], (((1,), (1,)), ((), ())),
        preferred_element_type=jnp.float32,
    )


def _matmul_chunk(w, xb, c, m_total, out_prev):
    seq, k = xb.shape
    chunk_m = w.shape[0] // k
    grid = (chunk_m // MM_BM,)
    blocks_before = c * (chunk_m // MM_BM)
    out_spec = pl.BlockSpec((MM_BM, seq), lambda i: (blocks_before + i, 0))
    in_specs = [
        pl.BlockSpec((MM_BM * k,), lambda i: (i,)),
        pl.BlockSpec((seq, k), lambda i: (0, 0)),
    ]
    out_shape = jax.ShapeDtypeStruct((m_total, seq), jnp.float32)
    if out_prev is None:
        return pl.pallas_call(
            functools.partial(_mm_first_body, k), grid=grid,
            in_specs=in_specs, out_specs=out_spec, out_shape=out_shape,
        )(w, xb)
    return pl.pallas_call(
        functools.partial(_mm_chain_body, k), grid=grid,
        in_specs=in_specs + [
            pl.BlockSpec(memory_space=pltpu.MemorySpace.HBM)],
        out_specs=out_spec, out_shape=out_shape,
        input_output_aliases={2: 0},
    )(w, xb, out_prev)


def kernel(x, values, row_indices, row_offsets, column_indices):
    b, seq, k = x.shape
    m = row_offsets.shape[0] - 1
    nnz_p = values.shape[0] // m

    r = column_indices & 31
    cols_p = (column_indices & ~31) | ((r & 1) << 4) | (r >> 1)

    xb = x[0].astype(jnp.bfloat16)

    chunk_m = m // NCHUNKS
    ws = [
        _densify_chunk(
            values, cols_p, nnz_p, chunk_m, k, c * chunk_m // GROUP_ROWS)
        for c in range(NCHUNKS)
    ]
    out = None
    for c in range(NCHUNKS):
        out = _matmul_chunk(ws[c], xb, c, m, out)
    return out.reshape(b, m, seq)


# speedup vs baseline: 1.2280x; 1.0686x over previous
import dataclasses
import functools

import jax
import jax.numpy as jnp
from jax import lax
from jax.experimental import pallas as pl
from jax.experimental.pallas import tpu as pltpu
from jax.experimental.pallas import tpu_sc as plsc

NUM_WORKERS = 32
LANES = 16
GROUP_ROWS = 16
CHUNK_WEIGHTS = (3, 2, 2, 1)
MM_BM = 512


def _densify_body(nnz_p, k, chunk_base_gid, groups,
                  vals_hbm, cols_hbm, w_hbm,
                  vals_v0, cols_v0, vals_v1, cols_v1, buf, bbuf_a, bbuf_b,
                  sem_a, sem_b, sem_in):
    wid = lax.axis_index("s") * 2 + lax.axis_index("c")
    group_nnz = nnz_p * GROUP_ROWS
    half = GROUP_ROWS * k // 2
    lane_base = lax.iota(jnp.int32, LANES) * k
    strip_base = lax.iota(jnp.int32, LANES) * nnz_p
    zeros16 = jnp.zeros((LANES,), jnp.float32)

    def fetch(gid, vals_v, cols_v):
        base = gid * group_nnz
        pltpu.async_copy(cols_hbm.at[pl.ds(base, group_nnz)], cols_v, sem_in)
        pltpu.async_copy(vals_hbm.at[pl.ds(base, group_nnz)], vals_v, sem_in)

    fetch(chunk_base_gid + wid * groups, vals_v0, cols_v0)

    @pl.loop(0, GROUP_ROWS * k, step=LANES, unroll=4)
    def _(j):
        buf[pl.ds(j, LANES)] = zeros16

    def convert_half(bbuf, offs):
        @pl.loop(0, half, step=2 * LANES, unroll=4)
        def _(j):
            a = buf[pl.ds(offs + j, LANES)]
            b = buf[pl.ds(offs + j + LANES, LANES)]
            bbuf[pl.ds(j, 2 * LANES)] = plsc.pack(
                a, b, format=plsc.PackFormat.INTERLEAVED)
            buf[pl.ds(offs + j, LANES)] = zeros16
            buf[pl.ds(offs + j + LANES, LANES)] = zeros16

    def handle(g, cur, nxt):
        vals_v, cols_v = cur
        local_gid = wid * groups + g
        gid = chunk_base_gid + local_gid
        pltpu.make_async_copy(
            cols_hbm.at[pl.ds(0, group_nnz)], cols_v, sem_in).wait()
        pltpu.make_async_copy(
            vals_hbm.at[pl.ds(0, group_nnz)], vals_v, sem_in).wait()

        if g + 1 < groups:
            fetch(gid + 1, *nxt)

        @pl.loop(0, nnz_p, unroll=4)
        def _(j):
            strip = strip_base + j
            cv = plsc.load_gather(cols_v, [strip])
            vv = plsc.load_gather(vals_v, [strip])
            plsc.addupdate_scatter(buf, [lane_base + cv], vv)

        out_base = local_gid * GROUP_ROWS * k

        if g > 0:
            pltpu.make_async_copy(
                bbuf_a, w_hbm.at[pl.ds(out_base, half)], sem_a).wait()
        convert_half(bbuf_a, 0)
        pltpu.async_copy(bbuf_a, w_hbm.at[pl.ds(out_base, half)], sem_a)

        if g > 0:
            pltpu.make_async_copy(
                bbuf_b, w_hbm.at[pl.ds(out_base + half, half)], sem_b).wait()
        convert_half(bbuf_b, half)
        pltpu.async_copy(bbuf_b, w_hbm.at[pl.ds(out_base + half, half)], sem_b)

    sets = ((vals_v0, cols_v0), (vals_v1, cols_v1))
    for g in range(groups):
        handle(g, sets[g % 2], sets[(g + 1) % 2])

    last = (wid * groups + groups - 1) * GROUP_ROWS * k
    pltpu.make_async_copy(bbuf_a, w_hbm.at[pl.ds(last, half)], sem_a).wait()
    pltpu.make_async_copy(
        bbuf_b, w_hbm.at[pl.ds(last + half, half)], sem_b).wait()


def _densify_chunk(values_g, cols_g, nnz_p, chunk_m, k, chunk_base_gid):
    groups = chunk_m // NUM_WORKERS // GROUP_ROWS
    mesh = plsc.VectorSubcoreMesh(core_axis_name="c", subcore_axis_name="s")
    cp = pltpu.CompilerParams()
    if "needs_layout_passes" in pltpu.CompilerParams.__dataclass_fields__:
        cp = dataclasses.replace(cp, needs_layout_passes=False)
    half = GROUP_ROWS * k // 2
    group_nnz = nnz_p * GROUP_ROWS
    kern = pl.kernel(
        functools.partial(_densify_body, nnz_p, k, chunk_base_gid, groups),
        out_type=jax.ShapeDtypeStruct((chunk_m * k,), jnp.bfloat16),
        mesh=mesh,
        scratch_types=[
            pltpu.VMEM((group_nnz,), jnp.float32),
            pltpu.VMEM((group_nnz,), jnp.int32),
            pltpu.VMEM((group_nnz,), jnp.float32),
            pltpu.VMEM((group_nnz,), jnp.int32),
            pltpu.VMEM((GROUP_ROWS * k,), jnp.float32),
            pltpu.VMEM((half,), jnp.bfloat16),
            pltpu.VMEM((half,), jnp.bfloat16),
            pltpu.SemaphoreType.DMA,
            pltpu.SemaphoreType.DMA,
            pltpu.SemaphoreType.DMA,
        ],
        compiler_params=cp,
    )
    return kern(values_g, cols_g)


def _mm_first_body(k, w_ref, xb_ref, o_ref):
    w = w_ref[...].reshape(MM_BM, k)
    o_ref[...] = lax.dot_general(
        w, xb_ref[...], (((1,), (1,)), ((), ())),
        preferred_element_type=jnp.float32,
    )


def _mm_chain_body(k, w_ref, xb_ref, prev_ref, o_ref):
    del prev_ref
    w = w_ref[...].reshape(MM_BM, k)
    o_ref[...] = lax.dot_general(
        w, xb_ref[...], (((1,), (1,)), ((), ())),
        preferred_element_type=jnp.float32,
    )


def _matmul_chunk(w, xb, row_base, m_total, out_prev):
    seq, k = xb.shape
    chunk_m = w.shape[0] // k
    grid = (chunk_m // MM_BM,)
    blocks_before = row_base // MM_BM
    out_spec = pl.BlockSpec((MM_BM, seq), lambda i: (blocks_before + i, 0))
    in_specs = [
        pl.BlockSpec((MM_BM * k,), lambda i: (i,)),
        pl.BlockSpec((seq, k), lambda i: (0, 0)),
    ]
    out_shape = jax.ShapeDtypeStruct((m_total, seq), jnp.float32)
    if out_prev is None:
        return pl.pallas_call(
            functools.partial(_mm_first_body, k), grid=grid,
            in_specs=in_specs, out_specs=out_spec, out_shape=out_shape,
        )(w, xb)
    return pl.pallas_call(
        functools.partial(_mm_chain_body, k), grid=grid,
        in_specs=in_specs + [
            pl.BlockSpec(memory_space=pltpu.MemorySpace.HBM)],
        out_specs=out_spec, out_shape=out_shape,
        input_output_aliases={2: 0},
    )(w, xb, out_prev)


def kernel(x, values, row_indices, row_offsets, column_indices):
    b, seq, k = x.shape
    m = row_offsets.shape[0] - 1
    nnz_p = values.shape[0] // m

    r = column_indices & 31
    cols_p = (column_indices & ~31) | ((r & 1) << 4) | (r >> 1)

    xb = x[0].astype(jnp.bfloat16)

    unit = NUM_WORKERS * GROUP_ROWS
    total_units = m // unit
    weights = (CHUNK_WEIGHTS if total_units == sum(CHUNK_WEIGHTS)
               else (total_units,))
    row_bases, sizes, base = [], [], 0
    for wgt in weights:
        row_bases.append(base)
        sizes.append(wgt * unit)
        base += wgt * unit
    ws = [
        _densify_chunk(
            values, cols_p, nnz_p, sizes[c], k, row_bases[c] // GROUP_ROWS)
        for c in range(len(sizes))
    ]
    out = None
    for c in range(len(sizes)):
        out = _matmul_chunk(ws[c], xb, row_bases[c], m, out)
    return out.reshape(b, m, seq)


# speedup vs baseline: 1.2476x; 1.0160x over previous
import dataclasses
import functools

import jax
import jax.numpy as jnp
from jax import lax
from jax.experimental import pallas as pl
from jax.experimental.pallas import tpu as pltpu
from jax.experimental.pallas import tpu_sc as plsc

NUM_WORKERS = 32
LANES = 16
GROUP_ROWS = 16
CHUNK_WEIGHTS = (4, 3, 1)
MM_BM = 512


def _densify_body(nnz_p, k, chunk_base_gid, groups,
                  vals_hbm, cols_hbm, w_hbm,
                  vals_v0, cols_v0, vals_v1, cols_v1, buf, bbuf_a, bbuf_b,
                  sem_a, sem_b, sem_in):
    wid = lax.axis_index("s") * 2 + lax.axis_index("c")
    group_nnz = nnz_p * GROUP_ROWS
    half = GROUP_ROWS * k // 2
    lane_base = lax.iota(jnp.int32, LANES) * k
    strip_base = lax.iota(jnp.int32, LANES) * nnz_p
    zeros16 = jnp.zeros((LANES,), jnp.float32)

    def fetch(gid, vals_v, cols_v):
        base = gid * group_nnz
        pltpu.async_copy(cols_hbm.at[pl.ds(base, group_nnz)], cols_v, sem_in)
        pltpu.async_copy(vals_hbm.at[pl.ds(base, group_nnz)], vals_v, sem_in)

    fetch(chunk_base_gid + wid * groups, vals_v0, cols_v0)

    @pl.loop(0, GROUP_ROWS * k, step=LANES, unroll=4)
    def _(j):
        buf[pl.ds(j, LANES)] = zeros16

    def convert_half(bbuf, offs):
        @pl.loop(0, half, step=2 * LANES, unroll=4)
        def _(j):
            a = buf[pl.ds(offs + j, LANES)]
            b = buf[pl.ds(offs + j + LANES, LANES)]
            bbuf[pl.ds(j, 2 * LANES)] = plsc.pack(
                a, b, format=plsc.PackFormat.INTERLEAVED)
            buf[pl.ds(offs + j, LANES)] = zeros16
            buf[pl.ds(offs + j + LANES, LANES)] = zeros16

    def handle(g, cur, nxt):
        vals_v, cols_v = cur
        local_gid = wid * groups + g
        gid = chunk_base_gid + local_gid
        pltpu.make_async_copy(
            cols_hbm.at[pl.ds(0, group_nnz)], cols_v, sem_in).wait()
        pltpu.make_async_copy(
            vals_hbm.at[pl.ds(0, group_nnz)], vals_v, sem_in).wait()

        if g + 1 < groups:
            fetch(gid + 1, *nxt)

        @pl.loop(0, nnz_p, unroll=4)
        def _(j):
            strip = strip_base + j
            cv = plsc.load_gather(cols_v, [strip])
            vv = plsc.load_gather(vals_v, [strip])
            plsc.addupdate_scatter(buf, [lane_base + cv], vv)

        out_base = local_gid * GROUP_ROWS * k

        if g > 0:
            pltpu.make_async_copy(
                bbuf_a, w_hbm.at[pl.ds(out_base, half)], sem_a).wait()
        convert_half(bbuf_a, 0)
        pltpu.async_copy(bbuf_a, w_hbm.at[pl.ds(out_base, half)], sem_a)

        if g > 0:
            pltpu.make_async_copy(
                bbuf_b, w_hbm.at[pl.ds(out_base + half, half)], sem_b).wait()
        convert_half(bbuf_b, half)
        pltpu.async_copy(bbuf_b, w_hbm.at[pl.ds(out_base + half, half)], sem_b)

    sets = ((vals_v0, cols_v0), (vals_v1, cols_v1))
    for g in range(groups):
        handle(g, sets[g % 2], sets[(g + 1) % 2])

    last = (wid * groups + groups - 1) * GROUP_ROWS * k
    pltpu.make_async_copy(bbuf_a, w_hbm.at[pl.ds(last, half)], sem_a).wait()
    pltpu.make_async_copy(
        bbuf_b, w_hbm.at[pl.ds(last + half, half)], sem_b).wait()


def _densify_chunk(values_g, cols_g, nnz_p, chunk_m, k, chunk_base_gid):
    groups = chunk_m // NUM_WORKERS // GROUP_ROWS
    mesh = plsc.VectorSubcoreMesh(core_axis_name="c", subcore_axis_name="s")
    cp = pltpu.CompilerParams()
    if "needs_layout_passes" in pltpu.CompilerParams.__dataclass_fields__:
        cp = dataclasses.replace(cp, needs_layout_passes=False)
    half = GROUP_ROWS * k // 2
    group_nnz = nnz_p * GROUP_ROWS
    kern = pl.kernel(
        functools.partial(_densify_body, nnz_p, k, chunk_base_gid, groups),
        out_type=jax.ShapeDtypeStruct((chunk_m * k,), jnp.bfloat16),
        mesh=mesh,
        scratch_types=[
            pltpu.VMEM((group_nnz,), jnp.float32),
            pltpu.VMEM((group_nnz,), jnp.int32),
            pltpu.VMEM((group_nnz,), jnp.float32),
            pltpu.VMEM((group_nnz,), jnp.int32),
            pltpu.VMEM((GROUP_ROWS * k,), jnp.float32),
            pltpu.VMEM((half,), jnp.bfloat16),
            pltpu.VMEM((half,), jnp.bfloat16),
            pltpu.SemaphoreType.DMA,
            pltpu.SemaphoreType.DMA,
            pltpu.SemaphoreType.DMA,
        ],
        compiler_params=cp,
    )
    return kern(values_g, cols_g)


def _mm_first_body(k, w_ref, xb_ref, o_ref):
    w = w_ref[...].reshape(MM_BM, k)
    o_ref[...] = lax.dot_general(
        w, xb_ref[...], (((1,), (1,)), ((), ())),
        preferred_element_type=jnp.float32,
    )


def _mm_chain_body(k, w_ref, xb_ref, prev_ref, o_ref):
    del prev_ref
    w = w_ref[...].reshape(MM_BM, k)
    o_ref[...] = lax.dot_general(
        w, xb_ref[...], (((1,), (1,)), ((), ())),
        preferred_element_type=jnp.float32,
    )


def _matmul_chunk(w, xb, row_base, m_total, out_prev):
    seq, k = xb.shape
    chunk_m = w.shape[0] // k
    grid = (chunk_m // MM_BM,)
    blocks_before = row_base // MM_BM
    out_spec = pl.BlockSpec((MM_BM, seq), lambda i: (blocks_before + i, 0))
    in_specs = [
        pl.BlockSpec((MM_BM * k,), lambda i: (i,)),
        pl.BlockSpec((seq, k), lambda i: (0, 0)),
    ]
    out_shape = jax.ShapeDtypeStruct((m_total, seq), jnp.float32)
    if out_prev is None:
        return pl.pallas_call(
            functools.partial(_mm_first_body, k), grid=grid,
            in_specs=in_specs, out_specs=out_spec, out_shape=out_shape,
        )(w, xb)
    return pl.pallas_call(
        functools.partial(_mm_chain_body, k), grid=grid,
        in_specs=in_specs + [
            pl.BlockSpec(memory_space=pltpu.MemorySpace.HBM)],
        out_specs=out_spec, out_shape=out_shape,
        input_output_aliases={2: 0},
    )(w, xb, out_prev)


def kernel(x, values, row_indices, row_offsets, column_indices):
    b, seq, k = x.shape
    m = row_offsets.shape[0] - 1
    nnz_p = values.shape[0] // m

    r = column_indices & 31
    cols_p = (column_indices & ~31) | ((r & 1) << 4) | (r >> 1)

    xb = x[0].astype(jnp.bfloat16)

    unit = NUM_WORKERS * GROUP_ROWS
    total_units = m // unit
    weights = (CHUNK_WEIGHTS if total_units == sum(CHUNK_WEIGHTS)
               else (total_units,))
    row_bases, sizes, base = [], [], 0
    for wgt in weights:
        row_bases.append(base)
        sizes.append(wgt * unit)
        base += wgt * unit
    ws = [
        _densify_chunk(
            values, cols_p, nnz_p, sizes[c], k, row_bases[c] // GROUP_ROWS)
        for c in range(len(sizes))
    ]
    out = None
    for c in range(len(sizes)):
        out = _matmul_chunk(ws[c], xb, row_bases[c], m, out)
    return out.reshape(b, m, seq)
